# baseline (speedup 1.0000x reference)
import functools

import jax
import jax.numpy as jnp
from jax.experimental import pallas as pl
from jax.experimental.pallas import tpu as pltpu
from jax.experimental.pallas import tpu_sc as plsc


_RATIO = 0.3
_RC = 256
_CJ = 512


def _stats_kernel(text_ref, out_ref):
    text = text_ref[...]
    B, L = text.shape
    m = jnp.max(text, axis=1, keepdims=True)
    ii = jax.lax.broadcasted_iota(jnp.int32, (B, L), 1)
    amax = jnp.min(jnp.where(text == m, ii, L), axis=1, keepdims=True)
    nval = jnp.sum((text != 0).astype(jnp.int32), axis=1, keepdims=True)
    out_ref[0:B, 0:1] = amax
    out_ref[0:B, 1:2] = nval


def _prefix_sum_row(x):
    L = x.shape[1]
    nchunks = L // _CJ
    r = jax.lax.broadcasted_iota(jnp.int32, (_CJ, _CJ), 0)
    c = jax.lax.broadcasted_iota(jnp.int32, (_CJ, _CJ), 1)
    tri = (r <= c).astype(jnp.float32)
    outs = []
    carry = jnp.zeros((1, 1), jnp.float32)
    for ci in range(nchunks):
        xc = x[:, ci * _CJ:(ci + 1) * _CJ]
        p = jnp.dot(xc, tri, preferred_element_type=jnp.float32)
        outs.append(p + carry)
        carry = carry + jnp.sum(xc, keepdims=True)
    return jnp.concatenate(outs, axis=1)


def _topk_kernel(s_ref, attnrow_ref, text_ref, idx_ref, *, B, L, k, kp):
    b = pl.program_id(0)
    amax_b = s_ref[b, 0]
    sub = amax_b - (amax_b // 8) * 8
    rsel = (jax.lax.broadcasted_iota(jnp.int32, (8, 1), 0) == sub)
    rself = rsel.astype(jnp.float32)

    row = jnp.sum(attnrow_ref[0] * rself, axis=0, keepdims=True)
    lane = jax.lax.broadcasted_iota(jnp.int32, (1, L), 1)
    row = jnp.where((lane == 0) | (lane == amax_b), -1.0, row)
    tmask = (text_ref[0, 0, :].reshape(1, L) != 0)
    row = row * tmask.astype(jnp.float32)
    row = row + 0.0

    ibits = jax.lax.bitcast_convert_type(row, jnp.int32)
    ubits = jax.lax.bitcast_convert_type(row, jnp.uint32)
    ukey = jnp.where(ibits < 0, ~ubits, ubits | jnp.uint32(0x80000000))

    def _bisect(i, t):
        bit = jnp.uint32(1) << (jnp.uint32(31) - i.astype(jnp.uint32))
        cand = t | bit
        cnt = jnp.sum((ukey >= cand).astype(jnp.int32))
        return jnp.where(cnt >= k, cand, t)

    tkey = jax.lax.fori_loop(0, 32, _bisect, jnp.uint32(0))
    gt = (ukey > tkey)
    eq = (ukey == tkey)
    c_gt = jnp.sum(gt.astype(jnp.int32))
    need = (k - c_gt).astype(jnp.float32)
    pref_eq = _prefix_sum_row(eq.astype(jnp.float32))
    keep = gt | (eq & (pref_eq <= need))
    keepf = keep.astype(jnp.float32)
    pos = _prefix_sum_row(keepf) - 1.0

    icol = jax.lax.broadcasted_iota(jnp.int32, (L, 1), 0)
    hi = (icol // 128).astype(jnp.float32)
    lo = (icol - (icol // 128) * 128).astype(jnp.float32)
    for rc in range(kp // _RC):
        pvals = (rc * _RC + jax.lax.broadcasted_iota(
            jnp.int32, (_RC, 1), 0)).astype(jnp.float32)
        selc = ((pos == pvals) & keep).astype(jnp.float32)
        ih = jnp.dot(selc, hi, preferred_element_type=jnp.float32)
        il = jnp.dot(selc, lo, preferred_element_type=jnp.float32)
        idxc = ih.astype(jnp.int32) * 128 + il.astype(jnp.int32) + b * L
        idx_ref[0, rc * _RC:(rc + 1) * _RC, 0:1] = idxc


def _mlp_kernel(s_ref, local_ref, gfrow_ref,
                w0t_ref, b0_ref, g0_ref, be0_ref, w1t_ref, b1_ref,
                lwt_ref, lb_ref, fused_ref, pooled_ref,
                h1_ref, sums_ref, *, B, D, E, k, kp):
    b = pl.program_id(0)
    amax_b = s_ref[b, 0]
    sub = amax_b - (amax_b // 8) * 8
    rsel = (jax.lax.broadcasted_iota(jnp.int32, (8, 1), 0) == sub)
    rself = rsel.astype(jnp.float32)

    gf = jnp.sum(gfrow_ref[0] * rself, axis=0, keepdims=True)
    gl = jnp.dot(gf, lwt_ref[...], preferred_element_type=jnp.float32)
    gl = gl + lb_ref[...].reshape(1, E)
    for bb in range(B):
        @pl.when(b == bb)
        def _():
            fused_ref[bb:bb + 1, :] = gl

    w0t = w0t_ref[...]
    b0 = b0_ref[...].reshape(1, D)
    ssum = jnp.zeros((1, D), jnp.float32)
    ssq = jnp.zeros((1, D), jnp.float32)
    for rc in range(kp // _RC):
        local = local_ref[0, rc * _RC:(rc + 1) * _RC, :]
        s2 = jnp.sum(local * local, axis=1, keepdims=True)
        local = local * (1.0 / (jnp.sqrt(s2) + 1e-8))
        h0 = jnp.dot(local, w0t, preferred_element_type=jnp.float32) + b0
        h1_ref[b, rc * _RC:(rc + 1) * _RC, :] = h0
        gmask = (rc * _RC + jax.lax.broadcasted_iota(
            jnp.int32, (_RC, 1), 0)) < k
        gm = gmask.astype(jnp.float32)
        ssum = ssum + jnp.sum(h0 * gm, axis=0, keepdims=True)
        ssq = ssq + jnp.sum(h0 * h0 * gm, axis=0, keepdims=True)

    @pl.when(b == 0)
    def _():
        sums_ref[0:1, :] = ssum
        sums_ref[1:2, :] = ssq

    @pl.when(b > 0)
    def _():
        sums_ref[0:1, :] = sums_ref[0:1, :] + ssum
        sums_ref[1:2, :] = sums_ref[1:2, :] + ssq

    @pl.when(b == B - 1)
    def _():
        n = float(B * k)
        mu = sums_ref[0:1, :] / n
        var = sums_ref[1:2, :] / n - mu * mu
        scale = g0_ref[...].reshape(1, D) * jax.lax.rsqrt(var + 1e-5)
        shift = be0_ref[...].reshape(1, D) - mu * scale
        w1t = w1t_ref[...]
        b1 = b1_ref[...].reshape(1, E)
        neg_inf = jnp.float32(-jnp.inf)
        for b2 in range(B):
            lens = jnp.minimum(s_ref[b2, 1] - 2, k)
            pooled = jnp.full((1, E), neg_inf, jnp.float32)
            for rc in range(kp // _RC):
                hc = h1_ref[b2, rc * _RC:(rc + 1) * _RC, :]
                a = jnp.maximum(hc * scale + shift, 0.0)
                h2 = jnp.dot(a, w1t, preferred_element_type=jnp.float32)
                rows = rc * _RC + jax.lax.broadcasted_iota(
                    jnp.int32, (_RC, 1), 0)
                h2 = jnp.where(rows < lens, h2, neg_inf)
                pooled = jnp.maximum(pooled, jnp.max(h2, axis=0,
                                                     keepdims=True))
            pooled = pooled + b1
            pooled_ref[b2:b2 + 1, :] = pooled
            fused_ref[b2:b2 + 1, :] = fused_ref[b2:b2 + 1, :] + pooled


def _sc_gather(table2d, idx_flat, nrows, D):
    NC, NS = 2, 16
    NW = NC * NS
    per = nrows // NW
    half = per // 2
    mesh = plsc.VectorSubcoreMesh(core_axis_name="c", subcore_axis_name="s",
                                  num_cores=NC)

    @functools.partial(
        pl.kernel, mesh=mesh,
        out_type=jax.ShapeDtypeStruct((nrows, D), jnp.float32),
        scratch_types=[
            pltpu.VMEM((half,), jnp.int32),
            pltpu.VMEM((half,), jnp.int32),
            pltpu.VMEM((per, D), jnp.float32),
            pltpu.SemaphoreType.DMA,
        ],
    )
    def k(table_hbm, idx_hbm, out_hbm, idx_v1, idx_v2, rows_v, sem):
        wid = jax.lax.axis_index("s") * NC + jax.lax.axis_index("c")
        base = wid * per
        pltpu.sync_copy(idx_hbm.at[pl.ds(base, half)], idx_v1)
        pltpu.sync_copy(idx_hbm.at[pl.ds(base + half, half)], idx_v2)
        cp1 = pltpu.async_copy(table_hbm.at[idx_v1],
                               rows_v.at[pl.ds(0, half)], sem)
        cp2 = pltpu.async_copy(table_hbm.at[idx_v2],
                               rows_v.at[pl.ds(half, half)], sem)
        cp1.wait()
        cp2.wait()
        pltpu.sync_copy(rows_v, out_hbm.at[pl.ds(base, per)])

    return k(table2d, idx_flat)


def kernel(gfeatures, features, text, atten, linear_W, linear_b,
           mlp_l0_W, mlp_l0_b, bn0_gamma, bn0_beta, mlp_l1_W, mlp_l1_b):
    del features
    B, L, D = gfeatures.shape
    E = linear_W.shape[0]
    k = int((L - 2) * _RATIO)
    kp = ((k + _RC - 1) // _RC) * _RC

    stats = pl.pallas_call(
        _stats_kernel,
        out_shape=jax.ShapeDtypeStruct((8, 128), jnp.int32),
    )(text)

    text3 = text.reshape(B, 1, L)

    topk_grid = pltpu.PrefetchScalarGridSpec(
        num_scalar_prefetch=1,
        grid=(B,),
        in_specs=[
            pl.BlockSpec((1, 8, L), lambda b, s: (b, s[b, 0] // 8, 0)),
            pl.BlockSpec((1, 1, L), lambda b, s: (b, 0, 0)),
        ],
        out_specs=pl.BlockSpec((1, kp, 1), lambda b, s: (b, 0, 0)),
    )
    idx3 = pl.pallas_call(
        functools.partial(_topk_kernel, B=B, L=L, k=k, kp=kp),
        grid_spec=topk_grid,
        out_shape=jax.ShapeDtypeStruct((B, kp, 1), jnp.int32),
        compiler_params=pltpu.CompilerParams(
            dimension_semantics=("arbitrary",)),
    )(stats, atten, text3)

    local_flat = _sc_gather(gfeatures.reshape(B * L, D),
                            idx3.reshape(B * kp), B * kp, D)
    local3 = local_flat.reshape(B, kp, D)

    w0t = mlp_l0_W.T
    w1t = mlp_l1_W.T
    lwt = linear_W.T

    mlp_grid = pltpu.PrefetchScalarGridSpec(
        num_scalar_prefetch=1,
        grid=(B,),
        in_specs=[
            pl.BlockSpec((1, kp, D), lambda b, s: (b, 0, 0)),
            pl.BlockSpec((1, 8, D), lambda b, s: (b, s[b, 0] // 8, 0)),
            pl.BlockSpec((D, D), lambda b, s: (0, 0)),
            pl.BlockSpec((D,), lambda b, s: (0,)),
            pl.BlockSpec((D,), lambda b, s: (0,)),
            pl.BlockSpec((D,), lambda b, s: (0,)),
            pl.BlockSpec((D, E), lambda b, s: (0, 0)),
            pl.BlockSpec((E,), lambda b, s: (0,)),
            pl.BlockSpec((D, E), lambda b, s: (0, 0)),
            pl.BlockSpec((E,), lambda b, s: (0,)),
        ],
        out_specs=[
            pl.BlockSpec((B, E), lambda b, s: (0, 0)),
            pl.BlockSpec((B, E), lambda b, s: (0, 0)),
        ],
        scratch_shapes=[
            pltpu.VMEM((B, kp, D), jnp.float32),
            pltpu.VMEM((8, D), jnp.float32),
        ],
    )
    fused, pooled = pl.pallas_call(
        functools.partial(_mlp_kernel, B=B, D=D, E=E, k=k, kp=kp),
        grid_spec=mlp_grid,
        out_shape=[
            jax.ShapeDtypeStruct((B, E), jnp.float32),
            jax.ShapeDtypeStruct((B, E), jnp.float32),
        ],
        compiler_params=pltpu.CompilerParams(
            dimension_semantics=("arbitrary",)),
    )(stats, local3, gfeatures, w0t, mlp_l0_b, bn0_gamma, bn0_beta,
      w1t, mlp_l1_b, lwt, linear_b)

    return (fused, pooled)


# speedup vs baseline: 1.3714x; 1.3714x over previous
import functools

import jax
import jax.numpy as jnp
from jax.experimental import pallas as pl
from jax.experimental.pallas import tpu as pltpu
from jax.experimental.pallas import tpu_sc as plsc


_RATIO = 0.3
_RC = 256
_CJ = 512


def _stats_kernel(text_ref, out_ref):
    text = text_ref[...]
    B, L = text.shape
    m = jnp.max(text, axis=1, keepdims=True)
    ii = jax.lax.broadcasted_iota(jnp.int32, (B, L), 1)
    amax = jnp.min(jnp.where(text == m, ii, L), axis=1, keepdims=True)
    nval = jnp.sum((text != 0).astype(jnp.int32), axis=1, keepdims=True)
    out_ref[0:B, 0:1] = amax
    out_ref[0:B, 1:2] = nval


def _prefix_sum_row(x):
    L = x.shape[1]
    nchunks = L // _CJ
    r = jax.lax.broadcasted_iota(jnp.int32, (_CJ, _CJ), 0)
    c = jax.lax.broadcasted_iota(jnp.int32, (_CJ, _CJ), 1)
    tri = (r <= c).astype(jnp.float32)
    outs = []
    carry = jnp.zeros((1, 1), jnp.float32)
    for ci in range(nchunks):
        xc = x[:, ci * _CJ:(ci + 1) * _CJ]
        p = jnp.dot(xc, tri, preferred_element_type=jnp.float32)
        outs.append(p + carry)
        carry = carry + jnp.sum(xc, keepdims=True)
    return jnp.concatenate(outs, axis=1)


def _topk_kernel(s_ref, attnrow_ref, text_ref, idx_ref, *, B, L, k, kp):
    b = pl.program_id(0)
    amax_b = s_ref[b, 0]
    sub = amax_b - (amax_b // 8) * 8
    rsel = (jax.lax.broadcasted_iota(jnp.int32, (8, 1), 0) == sub)
    rself = rsel.astype(jnp.float32)

    row = jnp.sum(attnrow_ref[0] * rself, axis=0, keepdims=True)
    lane = jax.lax.broadcasted_iota(jnp.int32, (1, L), 1)
    row = jnp.where((lane == 0) | (lane == amax_b), -1.0, row)
    tmask = (text_ref[0, 0, :].reshape(1, L) != 0)
    row = row * tmask.astype(jnp.float32)
    row = row + 0.0

    ibits = jax.lax.bitcast_convert_type(row, jnp.int32)
    ubits = jax.lax.bitcast_convert_type(row, jnp.uint32)
    ukey = jnp.where(ibits < 0, ~ubits, ubits | jnp.uint32(0x80000000))

    def _bisect(i, t):
        bit = jnp.uint32(1) << (jnp.uint32(31) - i.astype(jnp.uint32))
        cand = t | bit
        cnt = jnp.sum((ukey >= cand).astype(jnp.int32))
        return jnp.where(cnt >= k, cand, t)

    tkey = jax.lax.fori_loop(0, 32, _bisect, jnp.uint32(0))
    gt = (ukey > tkey)
    eq = (ukey == tkey)
    c_gt = jnp.sum(gt.astype(jnp.int32))
    need = (k - c_gt).astype(jnp.float32)
    pref_eq = _prefix_sum_row(eq.astype(jnp.float32))
    keep = gt | (eq & (pref_eq <= need))
    keepf = keep.astype(jnp.float32)
    pos = _prefix_sum_row(keepf) - 1.0

    pos_i = pos.astype(jnp.int32)
    nk = lane - pos_i - 1
    pad = kp - k
    dump = k + (nk - (nk // pad) * pad)
    bmod = b - (b // 2) * 2
    posg = jnp.where(keep, pos_i, dump) + bmod * kp
    idx_ref[0, 0:1, :] = posg


def _mlp_kernel(s_ref, local_ref, gfrow_ref,
                w0t_ref, b0_ref, g0_ref, be0_ref, w1t_ref, b1_ref,
                lwt_ref, lb_ref, fused_ref, pooled_ref,
                h1_ref, sums_ref, *, B, D, E, k, kp):
    b = pl.program_id(0)
    amax_b = s_ref[b, 0]
    sub = amax_b - (amax_b // 8) * 8
    rsel = (jax.lax.broadcasted_iota(jnp.int32, (8, 1), 0) == sub)
    rself = rsel.astype(jnp.float32)

    gf = jnp.sum(gfrow_ref[0] * rself, axis=0, keepdims=True)
    gl = jnp.dot(gf, lwt_ref[...], preferred_element_type=jnp.float32)
    gl = gl + lb_ref[...].reshape(1, E)
    for bb in range(B):
        @pl.when(b == bb)
        def _():
            fused_ref[bb:bb + 1, :] = gl

    w0t = w0t_ref[...]
    b0 = b0_ref[...].reshape(1, D)
    ssum = jnp.zeros((1, D), jnp.float32)
    ssq = jnp.zeros((1, D), jnp.float32)
    for rc in range(kp // _RC):
        local = local_ref[0, rc * _RC:(rc + 1) * _RC, :]
        s2 = jnp.sum(local * local, axis=1, keepdims=True)
        local = local * (1.0 / (jnp.sqrt(s2) + 1e-8))
        h0 = jnp.dot(local, w0t, preferred_element_type=jnp.float32) + b0
        h1_ref[b, rc * _RC:(rc + 1) * _RC, :] = h0
        gmask = (rc * _RC + jax.lax.broadcasted_iota(
            jnp.int32, (_RC, 1), 0)) < k
        gm = gmask.astype(jnp.float32)
        ssum = ssum + jnp.sum(h0 * gm, axis=0, keepdims=True)
        ssq = ssq + jnp.sum(h0 * h0 * gm, axis=0, keepdims=True)

    @pl.when(b == 0)
    def _():
        sums_ref[0:1, :] = ssum
        sums_ref[1:2, :] = ssq

    @pl.when(b > 0)
    def _():
        sums_ref[0:1, :] = sums_ref[0:1, :] + ssum
        sums_ref[1:2, :] = sums_ref[1:2, :] + ssq

    @pl.when(b == B - 1)
    def _():
        n = float(B * k)
        mu = sums_ref[0:1, :] / n
        var = sums_ref[1:2, :] / n - mu * mu
        scale = g0_ref[...].reshape(1, D) * jax.lax.rsqrt(var + 1e-5)
        shift = be0_ref[...].reshape(1, D) - mu * scale
        w1t = w1t_ref[...]
        b1 = b1_ref[...].reshape(1, E)
        neg_inf = jnp.float32(-jnp.inf)
        for b2 in range(B):
            lens = jnp.minimum(s_ref[b2, 1] - 2, k)
            pooled = jnp.full((1, E), neg_inf, jnp.float32)
            for rc in range(kp // _RC):
                hc = h1_ref[b2, rc * _RC:(rc + 1) * _RC, :]
                a = jnp.maximum(hc * scale + shift, 0.0)
                h2 = jnp.dot(a, w1t, preferred_element_type=jnp.float32)
                rows = rc * _RC + jax.lax.broadcasted_iota(
                    jnp.int32, (_RC, 1), 0)
                h2 = jnp.where(rows < lens, h2, neg_inf)
                pooled = jnp.maximum(pooled, jnp.max(h2, axis=0,
                                                     keepdims=True))
            pooled = pooled + b1
            pooled_ref[b2:b2 + 1, :] = pooled
            fused_ref[b2:b2 + 1, :] = fused_ref[b2:b2 + 1, :] + pooled


def _sc_scatter_gather(table2d, posg_flat, srcg_flat, B, L, kp, D):
    NC, NS = 2, 16
    nrows = B * kp
    per_e = (B * L) // (NC * NS)
    core_r = (B // NC) * kp
    per_r = nrows // (NC * NS)
    half = per_r // 2
    echunks = per_e // 128
    mesh = plsc.VectorSubcoreMesh(core_axis_name="c", subcore_axis_name="s",
                                  num_cores=NC)

    @functools.partial(
        pl.kernel, mesh=mesh,
        out_type=jax.ShapeDtypeStruct((nrows, D), jnp.float32),
        scratch_types=[
            pltpu.VMEM_SHARED((core_r,), jnp.int32),
            pltpu.VMEM((128,), jnp.int32),
            pltpu.VMEM((128,), jnp.int32),
            pltpu.VMEM((half,), jnp.int32),
            pltpu.VMEM((half,), jnp.int32),
            pltpu.VMEM((per_r, D), jnp.float32),
            pltpu.SemaphoreType.DMA,
        ],
    )
    def k(table_hbm, posg_hbm, srcg_hbm, out_hbm,
          sidx, pos_v, src_v, idx_v1, idx_v2, rows_v, sem):
        c = jax.lax.axis_index("c")
        s = jax.lax.axis_index("s")
        base_e = (c * NS + s) * per_e
        for j in range(echunks):
            off = base_e + j * 128
            pltpu.sync_copy(posg_hbm.at[pl.ds(off, 128)], pos_v)
            pltpu.sync_copy(srcg_hbm.at[pl.ds(off, 128)], src_v)
            pltpu.sync_copy(src_v, sidx.at[pos_v])
        plsc.subcore_barrier()
        srel = s * per_r
        pltpu.sync_copy(sidx.at[pl.ds(srel, half)], idx_v1)
        pltpu.sync_copy(sidx.at[pl.ds(srel + half, half)], idx_v2)
        cp1 = pltpu.async_copy(table_hbm.at[idx_v1],
                               rows_v.at[pl.ds(0, half)], sem)
        cp2 = pltpu.async_copy(table_hbm.at[idx_v2],
                               rows_v.at[pl.ds(half, half)], sem)
        cp1.wait()
        cp2.wait()
        base_r = (c * NS + s) * per_r
        pltpu.sync_copy(rows_v, out_hbm.at[pl.ds(base_r, per_r)])

    return k(table2d, posg_flat, srcg_flat)


def kernel(gfeatures, features, text, atten, linear_W, linear_b,
           mlp_l0_W, mlp_l0_b, bn0_gamma, bn0_beta, mlp_l1_W, mlp_l1_b):
    del features
    B, L, D = gfeatures.shape
    E = linear_W.shape[0]
    k = int((L - 2) * _RATIO)
    kp = ((k + _RC - 1) // _RC) * _RC

    stats = pl.pallas_call(
        _stats_kernel,
        out_shape=jax.ShapeDtypeStruct((8, 128), jnp.int32),
    )(text)

    text3 = text.reshape(B, 1, L)

    topk_grid = pltpu.PrefetchScalarGridSpec(
        num_scalar_prefetch=1,
        grid=(B,),
        in_specs=[
            pl.BlockSpec((1, 8, L), lambda b, s: (b, s[b, 0] // 8, 0)),
            pl.BlockSpec((1, 1, L), lambda b, s: (b, 0, 0)),
        ],
        out_specs=pl.BlockSpec((1, 1, L), lambda b, s: (b, 0, 0)),
    )
    posg3 = pl.pallas_call(
        functools.partial(_topk_kernel, B=B, L=L, k=k, kp=kp),
        grid_spec=topk_grid,
        out_shape=jax.ShapeDtypeStruct((B, 1, L), jnp.int32),
        compiler_params=pltpu.CompilerParams(
            dimension_semantics=("arbitrary",)),
    )(stats, atten, text3)

    srcg = jnp.arange(B * L, dtype=jnp.int32)
    local_flat = _sc_scatter_gather(gfeatures.reshape(B * L, D),
                                    posg3.reshape(B * L), srcg,
                                    B, L, kp, D)
    local3 = local_flat.reshape(B, kp, D)

    w0t = mlp_l0_W.T
    w1t = mlp_l1_W.T
    lwt = linear_W.T

    mlp_grid = pltpu.PrefetchScalarGridSpec(
        num_scalar_prefetch=1,
        grid=(B,),
        in_specs=[
            pl.BlockSpec((1, kp, D), lambda b, s: (b, 0, 0)),
            pl.BlockSpec((1, 8, D), lambda b, s: (b, s[b, 0] // 8, 0)),
            pl.BlockSpec((D, D), lambda b, s: (0, 0)),
            pl.BlockSpec((D,), lambda b, s: (0,)),
            pl.BlockSpec((D,), lambda b, s: (0,)),
            pl.BlockSpec((D,), lambda b, s: (0,)),
            pl.BlockSpec((D, E), lambda b, s: (0, 0)),
            pl.BlockSpec((E,), lambda b, s: (0,)),
            pl.BlockSpec((D, E), lambda b, s: (0, 0)),
            pl.BlockSpec((E,), lambda b, s: (0,)),
        ],
        out_specs=[
            pl.BlockSpec((B, E), lambda b, s: (0, 0)),
            pl.BlockSpec((B, E), lambda b, s: (0, 0)),
        ],
        scratch_shapes=[
            pltpu.VMEM((B, kp, D), jnp.float32),
            pltpu.VMEM((8, D), jnp.float32),
        ],
    )
    fused, pooled = pl.pallas_call(
        functools.partial(_mlp_kernel, B=B, D=D, E=E, k=k, kp=kp),
        grid_spec=mlp_grid,
        out_shape=[
            jax.ShapeDtypeStruct((B, E), jnp.float32),
            jax.ShapeDtypeStruct((B, E), jnp.float32),
        ],
        compiler_params=pltpu.CompilerParams(
            dimension_semantics=("arbitrary",)),
    )(stats, local3, gfeatures, w0t, mlp_l0_b, bn0_gamma, bn0_beta,
      w1t, mlp_l1_b, lwt, linear_b)

    return (fused, pooled)
